# baseline (speedup 1.0000x reference)
import functools

import jax
import jax.numpy as jnp
from jax import lax
from jax.experimental import pallas as pl
from jax.experimental.pallas import tpu as pltpu
from jax.experimental.pallas import tpu_sc as plsc

B = 16
BS = 8
N = 32
K = 4
H = 256
W = 256
QB = H // 4
SR = 32
Q = QB // SR
NC = 2
NS = 16


def _topk_idx_body(scores_ref, sc_ref, tc_ref):
    s16 = scores_ref[...]
    R = BS * 32

    def topk_picks(s):
        rows = s.shape[0]
        col = lax.broadcasted_iota(jnp.int32, (rows, N), 1)
        picks = []
        for _ in range(K):
            m = jnp.max(s, axis=1, keepdims=True)
            cand = jnp.where(s == m, col, N)
            amin = jnp.min(cand, axis=1, keepdims=True)
            picks.append(amin)
            s = jnp.where(col == amin, -jnp.inf, s)
        return picks

    s_rep = jnp.broadcast_to(s16[:BS, None, :], (BS, 32, N)).reshape(R, N)
    picks_sc = topk_picks(s_rep)
    row = lax.broadcasted_iota(jnp.int32, (R, SR), 0)
    j = lax.broadcasted_iota(jnp.int32, (R, SR), 1)
    b = row // 32
    hq = (row // 8) % 4
    q = (row // 4) % Q
    kk = row % K
    sel = jnp.zeros((R, SR), jnp.int32)
    for k in range(K):
        sel = sel + jnp.where(kk == k, picks_sc[k], 0)
    sc_ref[...] = (b * N + sel) * H + hq * QB + q * SR + j

    picks_tc = topk_picks(s16[BS:])
    tcol = lax.broadcasted_iota(jnp.int32, (B - BS, K), 1)
    tsel = jnp.zeros((B - BS, K), jnp.int32)
    for k in range(K):
        tsel = tsel + jnp.where(tcol == k, picks_tc[k], 0)
    tc_ref[...] = tsel


def _topk_idx(iou_scores):
    return pl.pallas_call(
        _topk_idx_body,
        out_shape=(
            jax.ShapeDtypeStruct((BS * 32, SR), jnp.int32),
            jax.ShapeDtypeStruct((B - BS, K), jnp.int32),
        ),
    )(iou_scores)


def _sc_gather_mean(idx, table):
    mesh = plsc.VectorSubcoreMesh(core_axis_name="c", subcore_axis_name="s")

    @functools.partial(
        pl.kernel,
        mesh=mesh,
        out_type=jax.ShapeDtypeStruct((B * H, W), jnp.float32),
        scratch_types=[
            pltpu.VMEM((Q * K, SR), jnp.int32),
            pltpu.VMEM((2, K, SR, W), jnp.float32),
            pltpu.VMEM((2, SR, W), jnp.float32),
            pltpu.SemaphoreType.DMA,
            pltpu.SemaphoreType.DMA,
            pltpu.SemaphoreType.DMA,
            pltpu.SemaphoreType.DMA,
        ],
    )
    def kern(idx_hbm, table_hbm, out_hbm, idx_v, stg, obuf, g0, g1, w0, w1):
        wid = lax.axis_index("s") * NC + lax.axis_index("c")
        b = wid // 4
        hq = wid % 4
        gsem = (g0, g1)
        wsem = (w0, w1)
        pltpu.sync_copy(idx_hbm.at[pl.ds(wid * (Q * K), Q * K)], idx_v)

        def gather(q, s):
            return [
                pltpu.async_copy(
                    table_hbm.at[idx_v.at[q * K + k]], stg.at[s, k], gsem[s])
                for k in range(K)
            ]

        gd = {0: gather(0, 0), 1: gather(1, 1)}
        wb = {}
        for q in range(Q):
            s = q % 2
            for c in gd.pop(q):
                c.wait()

            def body(i, _):
                for cc in range(W // 16):
                    sl = pl.ds(cc * 16, 16)
                    obuf[s, i, sl] = (
                        (stg[s, 0, i, sl] + stg[s, 1, i, sl])
                        + (stg[s, 2, i, sl] + stg[s, 3, i, sl])) * 0.25
                return 0

            lax.fori_loop(0, SR, body, 0)
            dst = out_hbm.at[pl.ds(b * H + hq * QB + q * SR, SR)]
            wb[q] = pltpu.async_copy(obuf.at[s], dst, wsem[s])
        for q in range(Q):
            wb.pop(q).wait()

    return kern(idx, table)


def _tc_gather_body(idx_ref, m0_ref, m1_ref, m2_ref, m3_ref, sc_ref, out_ref):
    del sc_ref
    out_ref[...] = ((m0_ref[...] + m1_ref[...])
                    + (m2_ref[...] + m3_ref[...])) * 0.25


def _tc_gather_mean(tc_idx, mask_preds, sc_out):
    def mk_spec(k):
        return pl.BlockSpec(
            (1, 1, H, W), lambda b, idx, k=k: (b + BS, idx[b * K + k], 0, 0))

    out_spec = pl.BlockSpec((1, 1, H, W), lambda b, idx: (b + BS, 0, 0, 0))
    grid_spec = pltpu.PrefetchScalarGridSpec(
        num_scalar_prefetch=1,
        grid=(B - BS,),
        in_specs=[mk_spec(k) for k in range(K)]
        + [pl.BlockSpec(memory_space=pl.ANY)],
        out_specs=out_spec,
    )
    return pl.pallas_call(
        _tc_gather_body,
        grid_spec=grid_spec,
        out_shape=jax.ShapeDtypeStruct((B, 1, H, W), jnp.float32),
        input_output_aliases={5: 0},
    )(tc_idx.reshape((B - BS) * K), mask_preds, mask_preds, mask_preds,
      mask_preds, sc_out)


def kernel(iou_scores, mask_preds):
    sc_idx, tc_idx = _topk_idx(iou_scores)
    table = mask_preds.reshape(B * N * H, W)
    sc_out = _sc_gather_mean(sc_idx, table)
    return _tc_gather_mean(tc_idx, mask_preds, sc_out.reshape(B, 1, H, W))


# speedup vs baseline: 1.0723x; 1.0723x over previous
import functools

import jax
import jax.numpy as jnp
from jax import lax
from jax.experimental import pallas as pl
from jax.experimental.pallas import tpu as pltpu
from jax.experimental.pallas import tpu_sc as plsc

B = 16
N = 32
K = 4
H = 256
W = 256
HB = H // 2
SR = 32
Q = HB // SR
NC = 2
NS = 16


def _sc_kernel(scores, table):
    mesh = plsc.VectorSubcoreMesh(core_axis_name="c", subcore_axis_name="s")

    @functools.partial(
        pl.kernel,
        mesh=mesh,
        compiler_params=pltpu.CompilerParams(needs_layout_passes=False),
        out_type=jax.ShapeDtypeStruct((B * H, W), jnp.float32),
        scratch_types=[
            pltpu.VMEM((N,), jnp.float32),
            pltpu.VMEM((Q * K, SR), jnp.int32),
            pltpu.VMEM((2, K, SR, W), jnp.float32),
            pltpu.VMEM((2, SR, W), jnp.float32),
            pltpu.SemaphoreType.DMA,
            pltpu.SemaphoreType.DMA,
            pltpu.SemaphoreType.DMA,
            pltpu.SemaphoreType.DMA,
        ],
    )
    def kern(sc_hbm, table_hbm, out_hbm, sv, idx_v, stg, obuf, g0, g1, w0, w1):
        wid = lax.axis_index("s") * NC + lax.axis_index("c")
        b = wid // 2
        h = wid % 2
        gsem = (g0, g1)
        wsem = (w0, w1)

        pltpu.sync_copy(sc_hbm.at[pl.ds(b * N, N)], sv)
        s0 = sv[pl.ds(0, 16)]
        s1 = sv[pl.ds(16, 16)]
        lanes = lax.iota(jnp.int32, 16)
        neg = jnp.full((16,), -jnp.inf, jnp.float32)

        def maxsplat(x):
            return plsc.cummax(lax.rev(plsc.cummax(x), (0,)))

        picks = []
        for _ in range(K):
            m = jnp.maximum(maxsplat(s0), maxsplat(s1))
            eq0 = s0 == m
            eq1 = s1 == m
            in0 = plsc.all_reduce_population_count(eq0) > 0
            f0 = plsc.all_reduce_ffs(eq0)
            f1 = plsc.all_reduce_ffs(eq1) + 16
            n = jnp.where(in0, f0, f1)
            picks.append(n)
            s0 = jnp.where(jnp.logical_and(in0, lanes == n), neg, s0)
            s1 = jnp.where(lanes == (n - 16), neg, s1)

        base = (b * N) * H + h * HB
        for q in range(Q):
            for k in range(K):
                v = base + picks[k] * H + (q * SR + lanes)
                idx_v[q * K + k, pl.ds(0, 16)] = v
                idx_v[q * K + k, pl.ds(16, 16)] = v + 16

        def gather(q, s):
            return [
                pltpu.async_copy(
                    table_hbm.at[idx_v.at[q * K + k]], stg.at[s, k], gsem[s])
                for k in range(K)
            ]

        gd = {0: gather(0, 0), 1: gather(1, 1)}
        wb = {}
        for q in range(Q):
            s = q % 2
            for c in gd.pop(q):
                c.wait()
            if q - 2 in wb:
                wb.pop(q - 2).wait()

            def body(i, _):
                for cc in range(W // 16):
                    sl = pl.ds(cc * 16, 16)
                    obuf[s, i, sl] = (
                        (stg[s, 0, i, sl] + stg[s, 1, i, sl])
                        + (stg[s, 2, i, sl] + stg[s, 3, i, sl])) * 0.25
                return 0

            lax.fori_loop(0, SR, body, 0)
            if q + 2 < Q:
                gd[q + 2] = gather(q + 2, s)
            dst = out_hbm.at[pl.ds(b * H + h * HB + q * SR, SR)]
            wb[q] = pltpu.async_copy(obuf.at[s], dst, wsem[s])
        for q in (Q - 2, Q - 1):
            wb.pop(q).wait()

    return kern(scores, table)


def kernel(iou_scores, mask_preds):
    table = mask_preds.reshape(B * N * H, W)
    out = _sc_kernel(iou_scores.reshape(B * N), table)
    return out.reshape(B, 1, H, W)


# speedup vs baseline: 1.0828x; 1.0098x over previous
import functools

import jax
import jax.numpy as jnp
from jax import lax
from jax.experimental import pallas as pl
from jax.experimental.pallas import tpu as pltpu
from jax.experimental.pallas import tpu_sc as plsc

B = 16
N = 32
K = 4
H = 256
W = 256
HB = H // 2
SR = 32
Q = HB // SR
NC = 2
NS = 16


def _sc_kernel(scores, table):
    mesh = plsc.VectorSubcoreMesh(core_axis_name="c", subcore_axis_name="s")

    @functools.partial(
        pl.kernel,
        mesh=mesh,
        compiler_params=pltpu.CompilerParams(needs_layout_passes=False),
        out_type=jax.ShapeDtypeStruct((B * H, W), jnp.float32),
        scratch_types=[
            pltpu.VMEM((N,), jnp.float32),
            pltpu.VMEM((Q * K, SR), jnp.int32),
            pltpu.VMEM((3, K, SR, W), jnp.float32),
            pltpu.VMEM((2, SR, W), jnp.float32),
            pltpu.SemaphoreType.DMA,
            pltpu.SemaphoreType.DMA,
            pltpu.SemaphoreType.DMA,
            pltpu.SemaphoreType.DMA,
            pltpu.SemaphoreType.DMA,
        ],
    )
    def kern(sc_hbm, table_hbm, out_hbm, sv, idx_v, stg, obuf, g0, g1, g2, w0, w1):
        wid = lax.axis_index("s") * NC + lax.axis_index("c")
        b = wid // 2
        h = wid % 2
        gsem = (g0, g1, g2)
        wsem = (w0, w1)

        pltpu.sync_copy(sc_hbm.at[pl.ds(b * N, N)], sv)
        s0 = sv[pl.ds(0, 16)]
        s1 = sv[pl.ds(16, 16)]
        lanes = lax.iota(jnp.int32, 16)
        neg = jnp.full((16,), -jnp.inf, jnp.float32)

        def maxsplat(x):
            return plsc.cummax(lax.rev(plsc.cummax(x), (0,)))

        picks = []
        for _ in range(K):
            m = jnp.maximum(maxsplat(s0), maxsplat(s1))
            eq0 = s0 == m
            eq1 = s1 == m
            in0 = plsc.all_reduce_population_count(eq0) > 0
            f0 = plsc.all_reduce_ffs(eq0)
            f1 = plsc.all_reduce_ffs(eq1) + 16
            n = jnp.where(in0, f0, f1)
            picks.append(n)
            s0 = jnp.where(jnp.logical_and(in0, lanes == n), neg, s0)
            s1 = jnp.where(lanes == (n - 16), neg, s1)

        base = (b * N) * H + h * HB
        for q in range(Q):
            for k in range(K):
                v = base + picks[k] * H + (q * SR + lanes)
                idx_v[q * K + k, pl.ds(0, 16)] = v
                idx_v[q * K + k, pl.ds(16, 16)] = v + 16

        def gather(q, s):
            return [
                pltpu.async_copy(
                    table_hbm.at[idx_v.at[q * K + k]], stg.at[s, k], gsem[s])
                for k in range(K)
            ]

        gd = {0: gather(0, 0), 1: gather(1, 1), 2: gather(2, 2)}
        wb = {}
        for q in range(Q):
            s = q % 3
            so = q % 2
            for c in gd.pop(q):
                c.wait()
            if q - 2 in wb:
                wb.pop(q - 2).wait()

            def body(i, _):
                for cc in range(W // 16):
                    sl = pl.ds(cc * 16, 16)
                    obuf[so, i, sl] = (
                        (stg[s, 0, i, sl] + stg[s, 1, i, sl])
                        + (stg[s, 2, i, sl] + stg[s, 3, i, sl])) * 0.25
                return 0

            lax.fori_loop(0, SR, body, 0)
            if q + 3 < Q:
                gd[q + 3] = gather(q + 3, s)
            dst = out_hbm.at[pl.ds(b * H + h * HB + q * SR, SR)]
            wb[q] = pltpu.async_copy(obuf.at[so], dst, wsem[so])
        for q in (Q - 2, Q - 1):
            wb.pop(q).wait()

    return kern(scores, table)


def kernel(iou_scores, mask_preds):
    table = mask_preds.reshape(B * N * H, W)
    out = _sc_kernel(iou_scores.reshape(B * N), table)
    return out.reshape(B, 1, H, W)
